# baseline (speedup 1.0000x reference)
import jax
import jax.numpy as jnp
from jax import lax
from jax.experimental import pallas as pl
from jax.experimental.pallas import tpu as pltpu
from jax.experimental.pallas import tpu_sc as plsc

VOCAB = 100000
D = 1024
B = 4
S = 2048
TOT = B * S

NC = 2
NS = 16
NW = NC * NS
PPW = S // NW
CC = 16
NSUB = PPW // CC
NCHUNK = NSUB * B
NB = 5
GAHEAD = 3
LANES = 16
VPR = D // LANES
HALF = (VPR // 2) * LANES


def _body(tok_hbm, idx_hbm, pos_hbm, out_hbm,
          idx_v, p0, p1, t0, t1, t2, t3, t4,
          sp0, sp1, si0, si1, si2, si3, si4, so0, so1, so2, so3, so4):
    w = lax.axis_index("s") * NC + lax.axis_index("c")
    pbase = w * PPW

    pbufs = (p0, p1)
    tbufs = (t0, t1, t2, t3, t4)
    psems = (sp0, sp1)
    isems = (si0, si1, si2, si3, si4)
    osems = (so0, so1, so2, so3, so4)

    for b in range(B):
        pltpu.sync_copy(idx_hbm.at[b, pl.ds(pbase, PPW)], idx_v.at[b])

    def start_pos(sub):
        pltpu.async_copy(pos_hbm.at[pl.ds(pbase + sub * CC, CC)],
                         pbufs[sub % 2], psems[sub % 2])

    def wait_pos(sub):
        pltpu.make_async_copy(pos_hbm.at[pl.ds(0, CC)], pbufs[sub % 2],
                              psems[sub % 2]).wait()

    def start_gather(c):
        sub, b = divmod(c, B)
        pltpu.async_copy(tok_hbm.at[idx_v.at[b, pl.ds(sub * CC, CC)]],
                         tbufs[c % NB], isems[c % NB])

    def wait_gather(c):
        pltpu.make_async_copy(tok_hbm.at[pl.ds(0, CC)], tbufs[c % NB],
                              isems[c % NB]).wait()

    def start_flush(c):
        sub, b = divmod(c, B)
        rbase = b * S + pbase + sub * CC
        pltpu.async_copy(tbufs[c % NB], out_hbm.at[pl.ds(rbase, CC)],
                         osems[c % NB])

    def wait_flush(c):
        pltpu.make_async_copy(tbufs[c % NB], out_hbm.at[pl.ds(0, CC)],
                              osems[c % NB]).wait()

    start_pos(0)
    start_pos(1)
    for c in range(GAHEAD):
        start_gather(c)

    for c in range(NCHUNK):
        sub, b = divmod(c, B)
        if c + GAHEAD < NCHUNK:
            if c + GAHEAD >= NB:
                wait_flush(c + GAHEAD - NB)
            start_gather(c + GAHEAD)
        wait_gather(c)
        if b == 0:
            wait_pos(sub)

        def add_row(r, carry, _k=c % NB, _pb=sub % 2):
            def add_q(q, carry2):
                j0 = q * (VPR // 4) * LANES
                for j in range(VPR // 4):
                    sl = pl.ds(j0 + j * LANES, LANES)
                    plsc.addupdate(tbufs[_k].at[r, sl], pbufs[_pb][r, sl])
                return carry2

            return lax.fori_loop(0, 4, add_q, carry)

        lax.fori_loop(0, CC, add_row, 0, unroll=False)

        if b == B - 1 and sub + 2 < NSUB:
            start_pos(sub + 2)
        start_flush(c)

    for c in range(NCHUNK - NB, NCHUNK):
        wait_flush(c)


@jax.jit
def _emb(tok_table, x2d, pos_table):
    mesh = plsc.VectorSubcoreMesh(core_axis_name="c", subcore_axis_name="s")
    return pl.kernel(
        _body,
        out_type=jax.ShapeDtypeStruct((TOT, D), jnp.float32),
        mesh=mesh,
        scratch_types=[
            pltpu.VMEM((B, PPW), jnp.int32),
            pltpu.VMEM((CC, D), jnp.float32),
            pltpu.VMEM((CC, D), jnp.float32),
            pltpu.VMEM((CC, D), jnp.float32),
            pltpu.VMEM((CC, D), jnp.float32),
            pltpu.VMEM((CC, D), jnp.float32),
            pltpu.VMEM((CC, D), jnp.float32),
            pltpu.VMEM((CC, D), jnp.float32),
            pltpu.SemaphoreType.DMA,
            pltpu.SemaphoreType.DMA,
            pltpu.SemaphoreType.DMA,
            pltpu.SemaphoreType.DMA,
            pltpu.SemaphoreType.DMA,
            pltpu.SemaphoreType.DMA,
            pltpu.SemaphoreType.DMA,
            pltpu.SemaphoreType.DMA,
            pltpu.SemaphoreType.DMA,
            pltpu.SemaphoreType.DMA,
            pltpu.SemaphoreType.DMA,
            pltpu.SemaphoreType.DMA,
        ],
    )(tok_table, x2d, pos_table)


def kernel(x, tok_table, pos_table):
    out = _emb(tok_table, x.astype(jnp.int32), pos_table)
    return out.reshape(B, S, D)


# speedup vs baseline: 1.8616x; 1.8616x over previous
import jax
import jax.numpy as jnp
from jax import lax
from jax.experimental import pallas as pl
from jax.experimental.pallas import tpu as pltpu
from jax.experimental.pallas import tpu_sc as plsc

VOCAB = 100000
D = 1024
B = 4
S = 2048
TOT = B * S

NC = 2
NS = 16
NW = NC * NS
PPW = S // NW
CC = 16
NSUB = PPW // CC
NCHUNK = NSUB * B
NB = 5
GAHEAD = 3
LANES = 16
VPR = D // LANES
AUNROLL = 8


def _body(tok_hbm, idx_hbm, pos_hbm, out_hbm,
          idx_v, p0, p1, t0, t1, t2, t3, t4,
          sp0, sp1, si0, si1, si2, si3, si4, so0, so1, so2, so3, so4):
    w = lax.axis_index("s") * NC + lax.axis_index("c")
    pbase = w * PPW

    pbufs = (p0, p1)
    tbufs = (t0, t1, t2, t3, t4)
    psems = (sp0, sp1)
    isems = (si0, si1, si2, si3, si4)
    osems = (so0, so1, so2, so3, so4)

    for b in range(B):
        pltpu.sync_copy(idx_hbm.at[b, pl.ds(pbase, PPW)], idx_v.at[b])

    def start_pos(sub):
        pltpu.async_copy(pos_hbm.at[pl.ds(pbase + sub * CC, CC)],
                         pbufs[sub % 2], psems[sub % 2])

    def wait_pos(sub):
        pltpu.make_async_copy(pos_hbm.at[pl.ds(0, CC)], pbufs[sub % 2],
                              psems[sub % 2]).wait()

    def start_gather(c):
        sub, b = divmod(c, B)
        pltpu.async_copy(tok_hbm.at[idx_v.at[b, pl.ds(sub * CC, CC)]],
                         tbufs[c % NB], isems[c % NB])

    def wait_gather(c):
        pltpu.make_async_copy(tok_hbm.at[pl.ds(0, CC)], tbufs[c % NB],
                              isems[c % NB]).wait()

    def start_flush(c):
        sub, b = divmod(c, B)
        rbase = b * S + pbase + sub * CC
        pltpu.async_copy(tbufs[c % NB], out_hbm.at[pl.ds(rbase, CC)],
                         osems[c % NB])

    def wait_flush(c):
        pltpu.make_async_copy(tbufs[c % NB], out_hbm.at[pl.ds(0, CC)],
                              osems[c % NB]).wait()

    start_pos(0)
    start_pos(1)
    for c in range(GAHEAD):
        start_gather(c)

    for c in range(NCHUNK):
        sub, b = divmod(c, B)
        if c + GAHEAD < NCHUNK:
            if c + GAHEAD >= NB:
                wait_flush(c + GAHEAD - NB)
            start_gather(c + GAHEAD)
        wait_gather(c)
        if b == 0:
            wait_pos(sub)

        tb, pb = tbufs[c % NB], pbufs[sub % 2]

        @plsc.parallel_loop(0, CC * VPR, step=1, unroll=AUNROLL)
        def _(i, _tb=tb, _pb=pb):
            r = i // VPR
            sl = pl.ds(lax.rem(i, VPR) * LANES, LANES)
            plsc.addupdate(_tb.at[r, sl], _pb[r, sl])

        if b == B - 1 and sub + 2 < NSUB:
            start_pos(sub + 2)
        start_flush(c)

    for c in range(NCHUNK - NB, NCHUNK):
        wait_flush(c)


@jax.jit
def _emb(tok_table, x2d, pos_table):
    mesh = plsc.VectorSubcoreMesh(core_axis_name="c", subcore_axis_name="s")
    return pl.kernel(
        _body,
        out_type=jax.ShapeDtypeStruct((TOT, D), jnp.float32),
        mesh=mesh,
        scratch_types=[
            pltpu.VMEM((B, PPW), jnp.int32),
            pltpu.VMEM((CC, D), jnp.float32),
            pltpu.VMEM((CC, D), jnp.float32),
            pltpu.VMEM((CC, D), jnp.float32),
            pltpu.VMEM((CC, D), jnp.float32),
            pltpu.VMEM((CC, D), jnp.float32),
            pltpu.VMEM((CC, D), jnp.float32),
            pltpu.VMEM((CC, D), jnp.float32),
            pltpu.SemaphoreType.DMA,
            pltpu.SemaphoreType.DMA,
            pltpu.SemaphoreType.DMA,
            pltpu.SemaphoreType.DMA,
            pltpu.SemaphoreType.DMA,
            pltpu.SemaphoreType.DMA,
            pltpu.SemaphoreType.DMA,
            pltpu.SemaphoreType.DMA,
            pltpu.SemaphoreType.DMA,
            pltpu.SemaphoreType.DMA,
            pltpu.SemaphoreType.DMA,
            pltpu.SemaphoreType.DMA,
        ],
    )(tok_table, x2d, pos_table)


def kernel(x, tok_table, pos_table):
    out = _emb(tok_table, x.astype(jnp.int32), pos_table)
    return out.reshape(B, S, D)


# speedup vs baseline: 1.8631x; 1.0008x over previous
import jax
import jax.numpy as jnp
from jax import lax
from jax.experimental import pallas as pl
from jax.experimental.pallas import tpu as pltpu
from jax.experimental.pallas import tpu_sc as plsc

VOCAB = 100000
D = 1024
B = 4
S = 2048
TOT = B * S

NC = 2
NS = 16
NW = NC * NS
PPW = S // NW
CC = 16
NSUB = PPW // CC
NCHUNK = NSUB * B
NB = 5
GAHEAD = 3
LANES = 16
VPR = D // LANES
AUNROLL = 4


def _body(tok_hbm, idx_hbm, pos_hbm, out_hbm,
          idx_v, p0, p1, t0, t1, t2, t3, t4,
          sp0, sp1, si0, si1, si2, si3, si4, so0, so1, so2, so3, so4):
    w = lax.axis_index("s") * NC + lax.axis_index("c")
    pbase = w * PPW

    pbufs = (p0, p1)
    tbufs = (t0, t1, t2, t3, t4)
    psems = (sp0, sp1)
    isems = (si0, si1, si2, si3, si4)
    osems = (so0, so1, so2, so3, so4)

    for b in range(B):
        pltpu.sync_copy(idx_hbm.at[b, pl.ds(pbase, PPW)], idx_v.at[b])

    def start_pos(sub):
        pltpu.async_copy(pos_hbm.at[pl.ds(pbase + sub * CC, CC)],
                         pbufs[sub % 2], psems[sub % 2])

    def wait_pos(sub):
        pltpu.make_async_copy(pos_hbm.at[pl.ds(0, CC)], pbufs[sub % 2],
                              psems[sub % 2]).wait()

    def start_gather(c):
        sub, b = divmod(c, B)
        pltpu.async_copy(tok_hbm.at[idx_v.at[b, pl.ds(sub * CC, CC)]],
                         tbufs[c % NB], isems[c % NB])

    def wait_gather(c):
        pltpu.make_async_copy(tok_hbm.at[pl.ds(0, CC)], tbufs[c % NB],
                              isems[c % NB]).wait()

    def start_flush(c):
        sub, b = divmod(c, B)
        rbase = b * S + pbase + sub * CC
        pltpu.async_copy(tbufs[c % NB], out_hbm.at[pl.ds(rbase, CC)],
                         osems[c % NB])

    def wait_flush(c):
        pltpu.make_async_copy(tbufs[c % NB], out_hbm.at[pl.ds(0, CC)],
                              osems[c % NB]).wait()

    start_pos(0)
    start_pos(1)
    for c in range(GAHEAD):
        start_gather(c)

    for c in range(NCHUNK):
        sub, b = divmod(c, B)
        if c + GAHEAD < NCHUNK:
            if c + GAHEAD >= NB:
                wait_flush(c + GAHEAD - NB)
            start_gather(c + GAHEAD)
        wait_gather(c)
        if b == 0:
            wait_pos(sub)

        tb, pb = tbufs[c % NB], pbufs[sub % 2]

        @plsc.parallel_loop(0, CC * VPR, step=1, unroll=AUNROLL)
        def _(i, _tb=tb, _pb=pb):
            r = i // VPR
            sl = pl.ds(lax.rem(i, VPR) * LANES, LANES)
            plsc.addupdate(_tb.at[r, sl], _pb[r, sl])

        if b == B - 1 and sub + 2 < NSUB:
            start_pos(sub + 2)
        start_flush(c)

    for c in range(NCHUNK - NB, NCHUNK):
        wait_flush(c)


@jax.jit
def _emb(tok_table, x2d, pos_table):
    mesh = plsc.VectorSubcoreMesh(core_axis_name="c", subcore_axis_name="s")
    return pl.kernel(
        _body,
        out_type=jax.ShapeDtypeStruct((TOT, D), jnp.float32),
        mesh=mesh,
        scratch_types=[
            pltpu.VMEM((B, PPW), jnp.int32),
            pltpu.VMEM((CC, D), jnp.float32),
            pltpu.VMEM((CC, D), jnp.float32),
            pltpu.VMEM((CC, D), jnp.float32),
            pltpu.VMEM((CC, D), jnp.float32),
            pltpu.VMEM((CC, D), jnp.float32),
            pltpu.VMEM((CC, D), jnp.float32),
            pltpu.VMEM((CC, D), jnp.float32),
            pltpu.SemaphoreType.DMA,
            pltpu.SemaphoreType.DMA,
            pltpu.SemaphoreType.DMA,
            pltpu.SemaphoreType.DMA,
            pltpu.SemaphoreType.DMA,
            pltpu.SemaphoreType.DMA,
            pltpu.SemaphoreType.DMA,
            pltpu.SemaphoreType.DMA,
            pltpu.SemaphoreType.DMA,
            pltpu.SemaphoreType.DMA,
            pltpu.SemaphoreType.DMA,
            pltpu.SemaphoreType.DMA,
        ],
    )(tok_table, x2d, pos_table)


def kernel(x, tok_table, pos_table):
    out = _emb(tok_table, x.astype(jnp.int32), pos_table)
    return out.reshape(B, S, D)


# speedup vs baseline: 1.8934x; 1.0162x over previous
import jax
import jax.numpy as jnp
from jax import lax
from jax.experimental import pallas as pl
from jax.experimental.pallas import tpu as pltpu
from jax.experimental.pallas import tpu_sc as plsc

VOCAB = 100000
D = 1024
B = 4
S = 2048
TOT = B * S

NC = 2
NS = 16
NW = NC * NS
PPW = S // NW
CC = 16
NSUB = PPW // CC
NCHUNK = NSUB * B
NB = 5
GAHEAD = 3
LANES = 16
VPR = D // LANES
AUNROLL = 4


def _body(tok_hbm, idx_hbm, pos_hbm, out_hbm,
          idx_v, p0, p1, t0, t1, t2, t3, t4,
          sp0, sp1, si0, si1, si2, si3, si4, so0, so1, so2, so3, so4):
    w = lax.axis_index("s") * NC + lax.axis_index("c")
    pbase = w * PPW

    pbufs = (p0, p1)
    tbufs = (t0, t1, t2, t3, t4)
    psems = (sp0, sp1)
    isems = (si0, si1, si2, si3, si4)
    osems = (so0, so1, so2, so3, so4)


    def start_pos(sub):
        pltpu.async_copy(pos_hbm.at[pl.ds(pbase + sub * CC, CC)],
                         pbufs[sub % 2], psems[sub % 2])

    def wait_pos(sub):
        pltpu.make_async_copy(pos_hbm.at[pl.ds(0, CC)], pbufs[sub % 2],
                              psems[sub % 2]).wait()

    def start_gather(c):
        sub, b = divmod(c, B)
        pltpu.async_copy(tok_hbm.at[idx_v.at[b, pl.ds(sub * CC, CC)]],
                         tbufs[c % NB], isems[c % NB])

    def wait_gather(c):
        pltpu.make_async_copy(tok_hbm.at[pl.ds(0, CC)], tbufs[c % NB],
                              isems[c % NB]).wait()

    def start_flush(c):
        sub, b = divmod(c, B)
        rbase = b * S + pbase + sub * CC
        pltpu.async_copy(tbufs[c % NB], out_hbm.at[pl.ds(rbase, CC)],
                         osems[c % NB])

    def wait_flush(c):
        pltpu.make_async_copy(tbufs[c % NB], out_hbm.at[pl.ds(0, CC)],
                              osems[c % NB]).wait()

    start_pos(0)
    start_pos(1)
    for b in range(B):
        pltpu.async_copy(idx_hbm.at[b, pl.ds(pbase, PPW)], idx_v.at[b],
                         isems[b])
    for b in range(B):
        pltpu.make_async_copy(idx_hbm.at[b, pl.ds(pbase, PPW)],
                              idx_v.at[b], isems[b]).wait()
    for c in range(GAHEAD):
        start_gather(c)

    for c in range(NCHUNK):
        sub, b = divmod(c, B)
        if c + GAHEAD < NCHUNK:
            if c + GAHEAD >= NB:
                wait_flush(c + GAHEAD - NB)
            start_gather(c + GAHEAD)
        wait_gather(c)
        if b == 0:
            wait_pos(sub)

        tb, pb = tbufs[c % NB], pbufs[sub % 2]

        @plsc.parallel_loop(0, CC * VPR, step=1, unroll=AUNROLL)
        def _(i, _tb=tb, _pb=pb):
            r = i // VPR
            sl = pl.ds(lax.rem(i, VPR) * LANES, LANES)
            plsc.addupdate(_tb.at[r, sl], _pb[r, sl])

        if b == B - 1 and sub + 2 < NSUB:
            start_pos(sub + 2)
        start_flush(c)

    for c in range(NCHUNK - NB, NCHUNK):
        wait_flush(c)


@jax.jit
def _emb(tok_table, x2d, pos_table):
    mesh = plsc.VectorSubcoreMesh(core_axis_name="c", subcore_axis_name="s")
    return pl.kernel(
        _body,
        out_type=jax.ShapeDtypeStruct((TOT, D), jnp.float32),
        mesh=mesh,
        scratch_types=[
            pltpu.VMEM((B, PPW), jnp.int32),
            pltpu.VMEM((CC, D), jnp.float32),
            pltpu.VMEM((CC, D), jnp.float32),
            pltpu.VMEM((CC, D), jnp.float32),
            pltpu.VMEM((CC, D), jnp.float32),
            pltpu.VMEM((CC, D), jnp.float32),
            pltpu.VMEM((CC, D), jnp.float32),
            pltpu.VMEM((CC, D), jnp.float32),
            pltpu.SemaphoreType.DMA,
            pltpu.SemaphoreType.DMA,
            pltpu.SemaphoreType.DMA,
            pltpu.SemaphoreType.DMA,
            pltpu.SemaphoreType.DMA,
            pltpu.SemaphoreType.DMA,
            pltpu.SemaphoreType.DMA,
            pltpu.SemaphoreType.DMA,
            pltpu.SemaphoreType.DMA,
            pltpu.SemaphoreType.DMA,
            pltpu.SemaphoreType.DMA,
            pltpu.SemaphoreType.DMA,
        ],
    )(tok_table, x2d, pos_table)


def kernel(x, tok_table, pos_table):
    out = _emb(tok_table, x.astype(jnp.int32), pos_table)
    return out.reshape(B, S, D)
